# baseline (speedup 1.0000x reference)
import functools

import jax
import jax.numpy as jnp
from jax import lax
from jax.experimental import pallas as pl
from jax.experimental.pallas import tpu as pltpu
from jax.experimental.pallas import tpu_sc as plsc

SEQ = 2048
HIDDEN = 64
NUM_CORES = 2
NUM_SUBCORES = 16
NUM_WORKERS = NUM_CORES * NUM_SUBCORES
HG = 8
LGROUPS = NUM_WORKERS // (HIDDEN // HG)


@functools.partial(jax.jit, static_argnums=(2, 3))
def _sc_gather(idx_t, table_t, batch, lookup):
  n_units = lookup // LGROUPS
  assert lookup % LGROUPS == 0 and n_units % 2 == 0 and batch % 16 == 0
  groups = batch // 16
  mesh = plsc.VectorSubcoreMesh(core_axis_name="c", subcore_axis_name="s")

  @functools.partial(
      pl.kernel,
      out_type=jax.ShapeDtypeStruct((lookup, HIDDEN, batch), jnp.float32),
      mesh=mesh,
      scratch_types=[
          pltpu.VMEM((HG, SEQ), jnp.float32),
          pltpu.VMEM((2, batch), jnp.int32),
          pltpu.VMEM((2, HG, batch), jnp.float32),
          [pltpu.SemaphoreType.DMA] * 2,
          [pltpu.SemaphoreType.DMA] * 2,
      ],
      compiler_params=pltpu.CompilerParams(needs_layout_passes=False),
  )
  def k(idx_hbm, table_hbm, out_hbm, table_v, idx_v, blk_v, isem, osem):
    wid = lax.axis_index("s") * NUM_CORES + lax.axis_index("c")
    hg = lax.rem(wid, HIDDEN // HG)
    lphase = wid // (HIDDEN // HG)
    pltpu.sync_copy(table_hbm.at[pl.ds(hg * HG, HG)], table_v)

    def lpos(k_):
      return lphase + k_ * LGROUPS

    def idx_start(k_, s):
      pltpu.async_copy(idx_hbm.at[lpos(k_)], idx_v.at[s], isem[s])

    def idx_wait(k_, s):
      pltpu.make_async_copy(idx_hbm.at[lpos(k_)], idx_v.at[s], isem[s]).wait()

    def store_start(k_, s):
      pltpu.async_copy(
          blk_v.at[s], out_hbm.at[lpos(k_), pl.ds(hg * HG, HG)], osem[s]
      )

    def store_wait(k_, s):
      pltpu.make_async_copy(
          blk_v.at[s], out_hbm.at[lpos(k_), pl.ds(hg * HG, HG)], osem[s]
      ).wait()

    h_vecs = [jnp.full((16,), h, jnp.int32) for h in range(HG)]

    def assemble(s):
      def body(g, _):
        seq = idx_v[s, pl.ds(g * 16, 16)]
        for h in range(HG):
          blk_v[s, h, pl.ds(g * 16, 16)] = plsc.load_gather(
              table_v, [h_vecs[h], seq]
          )
        return 0

      lax.fori_loop(0, groups, body, 0, unroll=8)

    def step(k_, s, first, last):
      if not last:
        idx_start(k_ + 1, 1 - s)
      idx_wait(k_, s)
      if not first:
        store_wait(k_ - 2 if k_ >= 2 else k_, s)
      assemble(s)
      store_start(k_, s)

    idx_start(0, 0)
    idx_start(1, 1)
    idx_wait(0, 0)
    assemble(0)
    store_start(0, 0)
    idx_start(2, 0)
    idx_wait(1, 1)
    assemble(1)
    store_start(1, 1)

    def group_body(p, _):
      k0 = p * 2
      idx_start(k0 + 1, 1)
      idx_wait(k0, 0)
      store_wait(k0 - 2, 0)
      assemble(0)
      store_start(k0, 0)
      idx_start(k0 + 2, 0)
      idx_wait(k0 + 1, 1)
      store_wait(k0 - 1, 1)
      assemble(1)
      store_start(k0 + 1, 1)
      return 0

    lax.fori_loop(1, n_units // 2 - 1, group_body, 0)

    k0 = n_units - 2
    idx_start(k0 + 1, 1)
    idx_wait(k0, 0)
    store_wait(k0 - 2, 0)
    assemble(0)
    store_start(k0, 0)
    idx_wait(k0 + 1, 1)
    store_wait(k0 - 1, 1)
    assemble(1)
    store_start(k0 + 1, 1)
    store_wait(k0, 0)
    store_wait(k0 + 1, 1)

  return k(idx_t, table_t)


def kernel(time_idx, positions):
  batch, lookup = time_idx.shape
  idx_t = time_idx.T
  table_t = positions.T
  out_t = _sc_gather(idx_t, table_t, batch, lookup)
  return jnp.transpose(out_t, (2, 0, 1))


# speedup vs baseline: 4.3892x; 4.3892x over previous
import functools

import jax
import jax.numpy as jnp
from jax import lax
from jax.experimental import pallas as pl
from jax.experimental.pallas import tpu as pltpu
from jax.experimental.pallas import tpu_sc as plsc

SEQ = 2048
HIDDEN = 64
NUM_CORES = 2
NUM_SUBCORES = 16
NUM_WORKERS = NUM_CORES * NUM_SUBCORES
HG = 8
LGROUPS = NUM_WORKERS // (HIDDEN // HG)


@functools.partial(jax.jit, static_argnums=(2, 3))
def _sc_gather(idx_t, table_t, batch, lookup):
  n_units = lookup // LGROUPS
  assert lookup % LGROUPS == 0 and n_units % 2 == 0 and batch % 16 == 0
  groups = batch // 16
  mesh = plsc.VectorSubcoreMesh(core_axis_name="c", subcore_axis_name="s")

  @functools.partial(
      pl.kernel,
      out_type=jax.ShapeDtypeStruct((lookup, HIDDEN, batch), jnp.float32),
      mesh=mesh,
      scratch_types=[
          pltpu.VMEM((HG, SEQ), jnp.float32),
          pltpu.VMEM((2, batch), jnp.int32),
          pltpu.VMEM((2, HG, batch), jnp.float32),
          [pltpu.SemaphoreType.DMA] * 2,
          [pltpu.SemaphoreType.DMA] * 2,
      ],
      compiler_params=pltpu.CompilerParams(needs_layout_passes=False),
  )
  def k(idx_hbm, table_hbm, out_hbm, table_v, idx_v, blk_v, isem, osem):
    wid = lax.axis_index("s") * NUM_CORES + lax.axis_index("c")
    hg = lax.rem(wid, HIDDEN // HG)
    lphase = wid // (HIDDEN // HG)
    pltpu.sync_copy(table_hbm.at[pl.ds(hg * HG, HG)], table_v)

    def lpos(k_):
      return lphase + k_ * LGROUPS

    def idx_start(k_, s):
      pltpu.async_copy(idx_hbm.at[lpos(k_)], idx_v.at[s], isem[s])

    def idx_wait(k_, s):
      pltpu.make_async_copy(idx_hbm.at[lpos(k_)], idx_v.at[s], isem[s]).wait()

    def store_start(k_, s):
      pltpu.async_copy(
          blk_v.at[s], out_hbm.at[lpos(k_), pl.ds(hg * HG, HG)], osem[s]
      )

    def store_wait(k_, s):
      pltpu.make_async_copy(
          blk_v.at[s], out_hbm.at[lpos(k_), pl.ds(hg * HG, HG)], osem[s]
      ).wait()

    h_vecs = [jnp.full((16,), h, jnp.int32) for h in range(HG)]

    def assemble(s):
      @plsc.parallel_loop(0, groups, unroll=4)
      def _(g):
        seq = idx_v[s, pl.ds(g * 16, 16)]
        for h in range(HG):
          blk_v[s, h, pl.ds(g * 16, 16)] = plsc.load_gather(
              table_v, [h_vecs[h], seq]
          )

    def step(k_, s, first, last):
      if not last:
        idx_start(k_ + 1, 1 - s)
      idx_wait(k_, s)
      if not first:
        store_wait(k_ - 2 if k_ >= 2 else k_, s)
      assemble(s)
      store_start(k_, s)

    idx_start(0, 0)
    idx_start(1, 1)
    idx_wait(0, 0)
    assemble(0)
    store_start(0, 0)
    idx_start(2, 0)
    idx_wait(1, 1)
    assemble(1)
    store_start(1, 1)

    def group_body(p, _):
      k0 = p * 2
      idx_start(k0 + 1, 1)
      idx_wait(k0, 0)
      store_wait(k0 - 2, 0)
      assemble(0)
      store_start(k0, 0)
      idx_start(k0 + 2, 0)
      idx_wait(k0 + 1, 1)
      store_wait(k0 - 1, 1)
      assemble(1)
      store_start(k0 + 1, 1)
      return 0

    lax.fori_loop(1, n_units // 2 - 1, group_body, 0)

    k0 = n_units - 2
    idx_start(k0 + 1, 1)
    idx_wait(k0, 0)
    store_wait(k0 - 2, 0)
    assemble(0)
    store_start(k0, 0)
    idx_wait(k0 + 1, 1)
    store_wait(k0 - 1, 1)
    assemble(1)
    store_start(k0 + 1, 1)
    store_wait(k0, 0)
    store_wait(k0 + 1, 1)

  return k(idx_t, table_t)


def kernel(time_idx, positions):
  batch, lookup = time_idx.shape
  idx_t = time_idx.T
  table_t = positions.T
  out_t = _sc_gather(idx_t, table_t, batch, lookup)
  return jnp.transpose(out_t, (2, 0, 1))


# speedup vs baseline: 4.3996x; 1.0024x over previous
import functools

import jax
import jax.numpy as jnp
from jax import lax
from jax.experimental import pallas as pl
from jax.experimental.pallas import tpu as pltpu
from jax.experimental.pallas import tpu_sc as plsc

SEQ = 2048
HIDDEN = 64
NUM_CORES = 2
NUM_SUBCORES = 16
NUM_WORKERS = NUM_CORES * NUM_SUBCORES
HG = 8
LGROUPS = NUM_WORKERS // (HIDDEN // HG)


@functools.partial(jax.jit, static_argnums=(2, 3))
def _sc_gather(idx_t, table_t, batch, lookup):
  n_units = lookup // LGROUPS
  assert lookup % LGROUPS == 0 and n_units % 2 == 0 and batch % 16 == 0
  groups = batch // 16
  mesh = plsc.VectorSubcoreMesh(core_axis_name="c", subcore_axis_name="s")

  @functools.partial(
      pl.kernel,
      out_type=jax.ShapeDtypeStruct((lookup, HIDDEN, batch), jnp.float32),
      mesh=mesh,
      scratch_types=[
          pltpu.VMEM((HG, SEQ), jnp.float32),
          pltpu.VMEM((2, batch), jnp.int32),
          pltpu.VMEM((2, HG, batch), jnp.float32),
          [pltpu.SemaphoreType.DMA] * 2,
          [pltpu.SemaphoreType.DMA] * 2,
      ],
      compiler_params=pltpu.CompilerParams(needs_layout_passes=False),
  )
  def k(idx_hbm, table_hbm, out_hbm, table_v, idx_v, blk_v, isem, osem):
    wid = lax.axis_index("s") * NUM_CORES + lax.axis_index("c")
    hg = lax.rem(wid, HIDDEN // HG)
    lphase = wid // (HIDDEN // HG)
    pltpu.sync_copy(table_hbm.at[pl.ds(hg * HG, HG)], table_v)

    def lpos(k_):
      return lphase + k_ * LGROUPS

    def idx_start(k_, s):
      pltpu.async_copy(idx_hbm.at[lpos(k_)], idx_v.at[s], isem[s])

    def idx_wait(k_, s):
      pltpu.make_async_copy(idx_hbm.at[lpos(k_)], idx_v.at[s], isem[s]).wait()

    def store_start(k_, s):
      pltpu.async_copy(
          blk_v.at[s], out_hbm.at[lpos(k_), pl.ds(hg * HG, HG)], osem[s]
      )

    def store_wait(k_, s):
      pltpu.make_async_copy(
          blk_v.at[s], out_hbm.at[lpos(k_), pl.ds(hg * HG, HG)], osem[s]
      ).wait()

    h_vecs = [jnp.full((16,), h, jnp.int32) for h in range(HG)]

    def assemble(s):
      @plsc.parallel_loop(0, groups, unroll=8)
      def _(g):
        seq = idx_v[s, pl.ds(g * 16, 16)]
        for h in range(HG):
          blk_v[s, h, pl.ds(g * 16, 16)] = plsc.load_gather(
              table_v, [h_vecs[h], seq]
          )

    def step(k_, s, first, last):
      if not last:
        idx_start(k_ + 1, 1 - s)
      idx_wait(k_, s)
      if not first:
        store_wait(k_ - 2 if k_ >= 2 else k_, s)
      assemble(s)
      store_start(k_, s)

    idx_start(0, 0)
    idx_start(1, 1)
    idx_wait(0, 0)
    assemble(0)
    store_start(0, 0)
    idx_start(2, 0)
    idx_wait(1, 1)
    assemble(1)
    store_start(1, 1)

    def group_body(p, _):
      k0 = p * 2
      idx_start(k0 + 1, 1)
      idx_wait(k0, 0)
      store_wait(k0 - 2, 0)
      assemble(0)
      store_start(k0, 0)
      idx_start(k0 + 2, 0)
      idx_wait(k0 + 1, 1)
      store_wait(k0 - 1, 1)
      assemble(1)
      store_start(k0 + 1, 1)
      return 0

    lax.fori_loop(1, n_units // 2 - 1, group_body, 0)

    k0 = n_units - 2
    idx_start(k0 + 1, 1)
    idx_wait(k0, 0)
    store_wait(k0 - 2, 0)
    assemble(0)
    store_start(k0, 0)
    idx_wait(k0 + 1, 1)
    store_wait(k0 - 1, 1)
    assemble(1)
    store_start(k0 + 1, 1)
    store_wait(k0, 0)
    store_wait(k0 + 1, 1)

  return k(idx_t, table_t)


def kernel(time_idx, positions):
  batch, lookup = time_idx.shape
  idx_t = time_idx.T
  table_t = positions.T
  out_t = _sc_gather(idx_t, table_t, batch, lookup)
  return jnp.transpose(out_t, (2, 0, 1))
